# Initial kernel scaffold; baseline (speedup 1.0000x reference)
import jax
import jax.numpy as jnp
from jax.experimental import pallas as pl


def kernel(batch_mask, mask_emb):
    raise NotImplementedError("write your pallas kernel here")


# speedup vs baseline: 7.1614x; 7.1614x over previous
import jax
import jax.numpy as jnp
from jax.experimental import pallas as pl


def _body(mask_ref, emb_ref, out_ref):
    m = mask_ref[...].astype(jnp.float32)
    e0 = emb_ref[0, :]
    d = emb_ref[1, :] - e0
    out_ref[...] = e0[None, None, :] + m[:, :, None] * d[None, None, :]


def kernel(batch_mask, mask_emb):
    M, N = batch_mask.shape
    _, D = mask_emb.shape
    BM = 128
    grid = M // BM
    return pl.pallas_call(
        _body,
        grid=(grid,),
        in_specs=[
            pl.BlockSpec((BM, N), lambda i: (i, 0)),
            pl.BlockSpec((2, D), lambda i: (0, 0)),
        ],
        out_specs=pl.BlockSpec((BM, N, D), lambda i: (i, 0, 0)),
        out_shape=jax.ShapeDtypeStruct((M, N, D), jnp.float32),
    )(batch_mask, mask_emb)
